# baseline (speedup 1.0000x reference)
import functools

import jax
import jax.numpy as jnp
from jax import lax
from jax.experimental import pallas as pl
from jax.experimental.pallas import tpu as pltpu
from jax.experimental.pallas import tpu_sc as plsc

NC = 2
NS = 16
NW = NC * NS
K = 80


def _rows_per_tile(n):
    rpt = -(-n // NS)
    return rpt + (-rpt) % K


def _sc_aggregate(table, src3d, dst3d, zero_rows):
    n, d = table.shape
    _, nch, _ = src3d.shape
    rpt = _rows_per_tile(n)
    nrch = rpt // K

    mesh = plsc.VectorSubcoreMesh(
        core_axis_name="c", subcore_axis_name="s",
        num_cores=NC, num_subcores=NS)

    def body(table_h, src_h, dst_h, zr_h, part_h,
             acc_sh, didx, sA, sB, rA, rB,
             semA, semB, semIA, semIB, semSA, semSB):
        c = lax.axis_index("c")
        s = lax.axis_index("s")
        wid = c * NS + s
        row0 = jnp.minimum(s * rpt, n - rpt)

        pltpu.sync_copy(dst_h.at[wid], didx)

        pltpu.sync_copy(zr_h, rA)

        def zloop(j, carry):
            pltpu.sync_copy(rA, acc_sh.at[pl.ds(row0 + j * K, K)])
            return carry

        lax.fori_loop(0, nrch, zloop, 0)
        plsc.subcore_barrier()

        bufA = (sA, rA, semA, semIA, semSA)
        bufB = (sB, rB, semB, semIB, semSB)

        def idx_start(g, buf):
            s_v, _, _, semI, _ = buf
            pltpu.async_copy(src_h.at[wid, g], s_v, semI)

        def gather_start(g, buf):
            s_v, r_v, sem, semI, _ = buf
            pltpu.make_async_copy(src_h.at[wid, g], s_v, semI).wait()
            pltpu.async_copy(table_h.at[s_v], r_v, sem)

        def gather_wait(buf):
            s_v, r_v, sem, _, _ = buf
            pltpu.make_async_copy(table_h.at[s_v], r_v, sem).wait()

        def scatter_start(g, buf):
            _, r_v, _, _, semS = buf
            pltpu.async_copy(r_v, acc_sh.at[didx.at[g]], semS, add=True)

        def scatter_wait(g, buf):
            _, r_v, _, _, semS = buf
            pltpu.make_async_copy(r_v, acc_sh.at[didx.at[g]], semS).wait()

        idx_start(0, bufA)
        idx_start(1, bufB)
        gather_start(0, bufA)
        gather_wait(bufA)
        scatter_start(0, bufA)
        idx_start(2, bufA)
        gather_start(1, bufB)

        def pipe(i, carry):
            g = 1 + 2 * i
            gather_wait(bufB)
            scatter_start(g, bufB)

            @pl.when(g + 2 < nch)
            def _():
                idx_start(g + 2, bufB)

            scatter_wait(g - 1, bufA)
            gather_start(g + 1, bufA)
            gather_wait(bufA)
            scatter_start(g + 1, bufA)

            @pl.when(g + 3 < nch)
            def _():
                idx_start(g + 3, bufA)

            scatter_wait(g, bufB)

            @pl.when(g + 2 < nch)
            def _():
                gather_start(g + 2, bufB)

            return carry

        lax.fori_loop(0, (nch - 1) // 2, pipe, 0)
        scatter_wait(nch - 1, bufA)

        plsc.subcore_barrier()

        def dump(j, carry):
            r = row0 + j * K
            pltpu.sync_copy(acc_sh.at[pl.ds(r, K)], rA)
            pltpu.sync_copy(rA, part_h.at[c, pl.ds(r, K)])
            return carry

        lax.fori_loop(0, nrch, dump, 0)

    kern = pl.kernel(
        body,
        out_type=jax.ShapeDtypeStruct((NC, n, d), jnp.float32),
        mesh=mesh,
        scratch_types=[
            pltpu.VMEM_SHARED((n, d), jnp.float32),
            pltpu.VMEM((nch, K), jnp.int32),
            pltpu.VMEM((K,), jnp.int32),
            pltpu.VMEM((K,), jnp.int32),
            pltpu.VMEM((K, d), jnp.float32),
            pltpu.VMEM((K, d), jnp.float32),
            pltpu.SemaphoreType.DMA,
            pltpu.SemaphoreType.DMA,
            pltpu.SemaphoreType.DMA,
            pltpu.SemaphoreType.DMA,
            pltpu.SemaphoreType.DMA,
            pltpu.SemaphoreType.DMA,
        ])
    return kern(table, src3d, dst3d, zero_rows)


def _tc_count(dst2d, n, interpret=False):
    e = dst2d.shape[0]
    eb = 4000
    hi_bins = -(-n // 128)

    def body(d_ref, o_ref):
        i = pl.program_id(0)

        @pl.when(i == 0)
        def _init():
            o_ref[...] = jnp.zeros_like(o_ref)

        dv = d_ref[...]
        lo = lax.rem(dv, 128)
        hi = lax.div(dv, 128)
        oh_lo = (lo == lax.broadcasted_iota(jnp.int32, (1, 128), 1)
                 ).astype(jnp.float32)
        oh_hi = (hi == lax.broadcasted_iota(jnp.int32, (1, hi_bins), 1)
                 ).astype(jnp.float32)
        dn = (((0,), (0,)), ((), ()))
        o_ref[...] += lax.dot_general(oh_hi, oh_lo, dn,
                                      preferred_element_type=jnp.float32)

    return pl.pallas_call(
        body,
        grid=(e // eb,),
        in_specs=[pl.BlockSpec((eb, 1), lambda i: (i, 0))],
        out_specs=pl.BlockSpec((hi_bins, 128), lambda i: (0, 0)),
        out_shape=jax.ShapeDtypeStruct((hi_bins, 128), jnp.float32),
        interpret=interpret,
    )(dst2d)


def _tc_sage(part, cnt, xin, wlT, wrT, brow, interpret=False):
    n, d = xin.shape
    r = 1000
    grid = (n // r,)

    def body(p_ref, c_ref, x_ref, wl_ref, wr_ref, b_ref, o_ref):
        p = p_ref[0] + p_ref[1]
        agg = p / jnp.maximum(c_ref[...], 1.0)
        h = (jnp.dot(agg, wl_ref[...], preferred_element_type=jnp.float32)
             + jnp.dot(x_ref[...], wr_ref[...],
                       preferred_element_type=jnp.float32)
             + b_ref[...])
        o_ref[...] = jnp.maximum(h, 0.0)

    return pl.pallas_call(
        body,
        grid=grid,
        in_specs=[
            pl.BlockSpec((NC, r, d), lambda i: (0, i, 0)),
            pl.BlockSpec((r, 1), lambda i: (i, 0)),
            pl.BlockSpec((r, d), lambda i: (i, 0)),
            pl.BlockSpec((d, d), lambda i: (0, 0)),
            pl.BlockSpec((d, d), lambda i: (0, 0)),
            pl.BlockSpec((1, d), lambda i: (0, 0)),
        ],
        out_specs=pl.BlockSpec((r, d), lambda i: (i, 0)),
        out_shape=jax.ShapeDtypeStruct((n, d), jnp.float32),
        interpret=interpret,
    )(part, cnt, xin, wlT, wrT, brow)


def _tc_final(part, cnt, h1, batch2d, pid2d, flow,
              w2lT, w2rT, b2row, wihT, whhT, bihrow, bhhrow,
              wfT, bfrow, ws1T, bs1row, ws2T, bs2row,
              n_graphs, path_len, interpret=False):
    n, d = h1.shape
    r = 1000
    g_steps = n // r
    npath = pid2d.shape[1]
    hh = wihT.shape[1] // 4

    def body(p_ref, c_ref, h1_ref, b_ref, pid_ref, flow_ref,
             w2l_ref, w2r_ref, b2_ref, wih_ref, whh_ref, bih_ref, bhh_ref,
             wf_ref, bf_ref, ws1_ref, bs1_ref, ws2_ref, bs2_ref,
             o_ref, pool_s, gcnt_s, prow_s):
        i = pl.program_id(0)

        @pl.when(i == 0)
        def _init():
            pool_s[...] = jnp.zeros_like(pool_s)
            gcnt_s[...] = jnp.zeros_like(gcnt_s)
            prow_s[...] = jnp.zeros_like(prow_s)

        p = p_ref[0] + p_ref[1]
        agg = p / jnp.maximum(c_ref[...], 1.0)
        h2 = (jnp.dot(agg, w2l_ref[...], preferred_element_type=jnp.float32)
              + jnp.dot(h1_ref[...], w2r_ref[...],
                        preferred_element_type=jnp.float32)
              + b2_ref[...])
        h2 = jnp.maximum(h2, 0.0)

        gids = lax.broadcasted_iota(jnp.int32, (1, n_graphs), 1)
        bm = (b_ref[...] == gids).astype(jnp.float32)
        dn = (((0,), (0,)), ((), ()))
        pool_s[...] += lax.dot_general(bm, h2, dn,
                                       preferred_element_type=jnp.float32)
        ones = jnp.ones((r, d), jnp.float32)
        gcnt_s[...] += lax.dot_general(bm, ones, dn,
                                       preferred_element_type=jnp.float32)

        rid = lax.broadcasted_iota(jnp.int32, (r, 1), 0) + i * r
        pm = (rid == pid_ref[...]).astype(jnp.float32)
        prow_s[...] += lax.dot_general(pm, h2, dn,
                                       preferred_element_type=jnp.float32)

        @pl.when(i == g_steps - 1)
        def _epilogue():
            graph_emb = pool_s[...] / jnp.maximum(gcnt_s[...], 1.0)
            flow_emb = jnp.maximum(
                jnp.dot(flow_ref[...], wf_ref[...],
                        preferred_element_type=jnp.float32) + bf_ref[...],
                0.0)
            rows = prow_s[...]
            h = jnp.zeros((n_graphs, hh), jnp.float32)
            c = jnp.zeros((n_graphs, hh), jnp.float32)
            for t in range(path_len):
                xt = rows[t * n_graphs:(t + 1) * n_graphs, :]
                gg = (jnp.dot(xt, wih_ref[...],
                              preferred_element_type=jnp.float32)
                      + bih_ref[...]
                      + jnp.dot(h, whh_ref[...],
                                preferred_element_type=jnp.float32)
                      + bhh_ref[...])
                gi = jax.nn.sigmoid(gg[:, 0 * hh:1 * hh])
                gf = jax.nn.sigmoid(gg[:, 1 * hh:2 * hh])
                gc = jnp.tanh(gg[:, 2 * hh:3 * hh])
                go = jax.nn.sigmoid(gg[:, 3 * hh:4 * hh])
                c = gf * c + gi * gc
                h = go * jnp.tanh(c)
            comb = jnp.concatenate([graph_emb, h, flow_emb], axis=1)
            hid = jnp.maximum(
                jnp.dot(comb, ws1_ref[...],
                        preferred_element_type=jnp.float32) + bs1_ref[...],
                0.0)
            o_ref[...] = (jnp.dot(hid, ws2_ref[...],
                                  preferred_element_type=jnp.float32)
                          + bs2_ref[...])

    whole = lambda shape: pl.BlockSpec(shape, lambda i: tuple(0 for _ in shape))
    return pl.pallas_call(
        body,
        grid=(g_steps,),
        in_specs=[
            pl.BlockSpec((NC, r, d), lambda i: (0, i, 0)),
            pl.BlockSpec((r, 1), lambda i: (i, 0)),
            pl.BlockSpec((r, d), lambda i: (i, 0)),
            pl.BlockSpec((r, 1), lambda i: (i, 0)),
            whole(pid2d.shape),
            whole(flow.shape),
            whole(w2lT.shape),
            whole(w2rT.shape),
            whole(b2row.shape),
            whole(wihT.shape),
            whole(whhT.shape),
            whole(bihrow.shape),
            whole(bhhrow.shape),
            whole(wfT.shape),
            whole(bfrow.shape),
            whole(ws1T.shape),
            whole(bs1row.shape),
            whole(ws2T.shape),
            whole(bs2row.shape),
        ],
        out_specs=pl.BlockSpec((n_graphs, 1), lambda i: (0, 0)),
        out_shape=jax.ShapeDtypeStruct((n_graphs, 1), jnp.float32),
        scratch_shapes=[
            pltpu.VMEM((n_graphs, d), jnp.float32),
            pltpu.VMEM((n_graphs, d), jnp.float32),
            pltpu.VMEM((npath, d), jnp.float32),
        ],
        interpret=interpret,
    )(part, cnt, h1, batch2d, pid2d, flow,
      w2lT, w2rT, b2row, wihT, whhT, bihrow, bhhrow,
      wfT, bfrow, ws1T, bs1row, ws2T, bs2row)


def kernel(x, edge_index, batch, path_node_indices_list, flow_feature_batch,
           W1l, W1r, b1, W2l, W2r, b2, W_ih, W_hh, b_ih, b_hh,
           Wf, bf, Ws1, bs1, Ws2, bs2):
    n, d = x.shape
    n_graphs, path_len = path_node_indices_list.shape
    src = edge_index[0]
    dst = edge_index[1]
    e = src.shape[0]
    nch = e // (NW * K)
    src3d = src.reshape(NW, nch, K)
    dst3d = dst.reshape(NW, nch, K)

    zero_rows = jnp.zeros((K, d), jnp.float32)

    cnt = _tc_count(dst.reshape(-1, 1), n)
    cnt = cnt.reshape(-1)[:n].reshape(n, 1)

    part1 = _sc_aggregate(x, src3d, dst3d, zero_rows)
    h1 = _tc_sage(part1, cnt, x, W1l.T, W1r.T, b1.reshape(1, -1))
    part2 = _sc_aggregate(h1, src3d, dst3d, zero_rows)

    pid2d = path_node_indices_list.T.reshape(1, -1)
    scores = _tc_final(
        part2, cnt, h1, batch.reshape(-1, 1), pid2d, flow_feature_batch,
        W2l.T, W2r.T, b2.reshape(1, -1),
        W_ih.T, W_hh.T, b_ih.reshape(1, -1), b_hh.reshape(1, -1),
        Wf.T, bf.reshape(1, -1), Ws1.T, bs1.reshape(1, -1),
        Ws2.T, bs2.reshape(1, 1),
        n_graphs, path_len)
    return scores.reshape(-1)


# speedup vs baseline: 1.0440x; 1.0440x over previous
import functools

import jax
import jax.numpy as jnp
from jax import lax
from jax.experimental import pallas as pl
from jax.experimental.pallas import tpu as pltpu
from jax.experimental.pallas import tpu_sc as plsc

NC = 2
NS = 16
NW = NC * NS
K = 80


def _rows_per_tile(n):
    rpt = -(-n // NS)
    return rpt + (-rpt) % K


def _sc_aggregate(table, src3d, dst3d, zero_rows):
    n, d = table.shape
    _, nch, _ = src3d.shape
    rpt = _rows_per_tile(n)
    nrch = rpt // K

    mesh = plsc.VectorSubcoreMesh(
        core_axis_name="c", subcore_axis_name="s",
        num_cores=NC, num_subcores=NS)

    def body(table_h, src_h, dst_h, zr_h, part_h,
             acc_sh, didx, sA, sB, rA, rB,
             semA, semB, semIA, semIB, semSA, semSB):
        c = lax.axis_index("c")
        s = lax.axis_index("s")
        wid = c * NS + s
        row0 = jnp.minimum(s * rpt, n - rpt)

        pltpu.sync_copy(dst_h.at[wid], didx)

        pltpu.sync_copy(zr_h, rA)

        def zloop(j, carry):
            pltpu.sync_copy(rA, acc_sh.at[pl.ds(row0 + j * K, K)])
            return carry

        lax.fori_loop(0, nrch, zloop, 0)
        plsc.subcore_barrier()

        bufA = (sA, rA, semA, semIA, semSA)
        bufB = (sB, rB, semB, semIB, semSB)

        def idx_start(g, buf):
            s_v, _, _, semI, _ = buf
            pltpu.async_copy(src_h.at[wid, g], s_v, semI)

        def gather_start(g, buf):
            s_v, r_v, sem, semI, _ = buf
            pltpu.make_async_copy(src_h.at[wid, g], s_v, semI).wait()
            pltpu.async_copy(table_h.at[s_v], r_v, sem)

        def gather_wait(buf):
            s_v, r_v, sem, _, _ = buf
            pltpu.make_async_copy(table_h.at[s_v], r_v, sem).wait()

        def scatter(g, buf):
            _, r_v, _, _, _ = buf
            pltpu.sync_copy(r_v, acc_sh.at[didx.at[g]], add=True)

        idx_start(0, bufA)
        idx_start(1, bufB)
        gather_start(0, bufA)

        def pipe(i, carry):
            g = 1 + 2 * i
            gather_start(g, bufB)
            gather_wait(bufA)
            idx_start(g + 1, bufA)
            scatter(g - 1, bufA)
            gather_start(g + 1, bufA)
            gather_wait(bufB)

            @pl.when(g + 2 < nch)
            def _():
                idx_start(g + 2, bufB)

            scatter(g, bufB)
            return carry

        lax.fori_loop(0, (nch - 1) // 2, pipe, 0)
        gather_wait(bufA)
        scatter(nch - 1, bufA)

        plsc.subcore_barrier()

        def dump(j, carry):
            r = row0 + j * K
            pltpu.sync_copy(acc_sh.at[pl.ds(r, K)], rA)
            pltpu.sync_copy(rA, part_h.at[c, pl.ds(r, K)])
            return carry

        lax.fori_loop(0, nrch, dump, 0)

    kern = pl.kernel(
        body,
        out_type=jax.ShapeDtypeStruct((NC, n, d), jnp.float32),
        mesh=mesh,
        scratch_types=[
            pltpu.VMEM_SHARED((n, d), jnp.float32),
            pltpu.VMEM((nch, K), jnp.int32),
            pltpu.VMEM((K,), jnp.int32),
            pltpu.VMEM((K,), jnp.int32),
            pltpu.VMEM((K, d), jnp.float32),
            pltpu.VMEM((K, d), jnp.float32),
            pltpu.SemaphoreType.DMA,
            pltpu.SemaphoreType.DMA,
            pltpu.SemaphoreType.DMA,
            pltpu.SemaphoreType.DMA,
            pltpu.SemaphoreType.DMA,
            pltpu.SemaphoreType.DMA,
        ])
    return kern(table, src3d, dst3d, zero_rows)


def _tc_count(dst2d, n, interpret=False):
    e = dst2d.shape[0]
    eb = 4000
    hi_bins = -(-n // 128)

    def body(d_ref, o_ref):
        i = pl.program_id(0)

        @pl.when(i == 0)
        def _init():
            o_ref[...] = jnp.zeros_like(o_ref)

        dv = d_ref[...]
        lo = lax.rem(dv, 128)
        hi = lax.div(dv, 128)
        oh_lo = (lo == lax.broadcasted_iota(jnp.int32, (1, 128), 1)
                 ).astype(jnp.float32)
        oh_hi = (hi == lax.broadcasted_iota(jnp.int32, (1, hi_bins), 1)
                 ).astype(jnp.float32)
        dn = (((0,), (0,)), ((), ()))
        o_ref[...] += lax.dot_general(oh_hi, oh_lo, dn,
                                      preferred_element_type=jnp.float32)

    return pl.pallas_call(
        body,
        grid=(e // eb,),
        in_specs=[pl.BlockSpec((eb, 1), lambda i: (i, 0))],
        out_specs=pl.BlockSpec((hi_bins, 128), lambda i: (0, 0)),
        out_shape=jax.ShapeDtypeStruct((hi_bins, 128), jnp.float32),
        interpret=interpret,
    )(dst2d)


def _tc_xr(xin, wrT, brow, interpret=False):
    n, d = xin.shape
    r = 1000

    def body(x_ref, wr_ref, b_ref, o_ref):
        o_ref[...] = (jnp.dot(x_ref[...], wr_ref[...],
                              preferred_element_type=jnp.float32)
                      + b_ref[...])

    return pl.pallas_call(
        body,
        grid=(n // r,),
        in_specs=[
            pl.BlockSpec((r, d), lambda i: (i, 0)),
            pl.BlockSpec((d, d), lambda i: (0, 0)),
            pl.BlockSpec((1, d), lambda i: (0, 0)),
        ],
        out_specs=pl.BlockSpec((r, d), lambda i: (i, 0)),
        out_shape=jax.ShapeDtypeStruct((n, d), jnp.float32),
        interpret=interpret,
    )(xin, wrT, brow)


def _tc_sage(part, cnt, xr, wlT, interpret=False):
    n, d = xr.shape
    r = 1000

    def body(p_ref, c_ref, xr_ref, wl_ref, o_ref):
        p = p_ref[0] + p_ref[1]
        agg = p / jnp.maximum(c_ref[...], 1.0)
        h = (jnp.dot(agg, wl_ref[...], preferred_element_type=jnp.float32)
             + xr_ref[...])
        o_ref[...] = jnp.maximum(h, 0.0)

    return pl.pallas_call(
        body,
        grid=(n // r,),
        in_specs=[
            pl.BlockSpec((NC, r, d), lambda i: (0, i, 0)),
            pl.BlockSpec((r, 1), lambda i: (i, 0)),
            pl.BlockSpec((r, d), lambda i: (i, 0)),
            pl.BlockSpec((d, d), lambda i: (0, 0)),
        ],
        out_specs=pl.BlockSpec((r, d), lambda i: (i, 0)),
        out_shape=jax.ShapeDtypeStruct((n, d), jnp.float32),
        interpret=interpret,
    )(part, cnt, xr, wlT)


def _tc_final(part, cnt, h1r, batch2d, pid2d, flow,
              w2lT, wihT, whhT, bihrow, bhhrow,
              wfT, bfrow, ws1T, bs1row, ws2T, bs2row,
              n_graphs, path_len, interpret=False):
    n, d = h1r.shape
    r = 1000
    g_steps = n // r
    npath = pid2d.shape[1]
    hh = wihT.shape[1] // 4

    def body(p_ref, c_ref, h1r_ref, b_ref, pid_ref, flow_ref,
             w2l_ref, wih_ref, whh_ref, bih_ref, bhh_ref,
             wf_ref, bf_ref, ws1_ref, bs1_ref, ws2_ref, bs2_ref,
             o_ref, pool_s, gcnt_s, prow_s):
        i = pl.program_id(0)

        @pl.when(i == 0)
        def _init():
            pool_s[...] = jnp.zeros_like(pool_s)
            gcnt_s[...] = jnp.zeros_like(gcnt_s)
            prow_s[...] = jnp.zeros_like(prow_s)

        p = p_ref[0] + p_ref[1]
        agg = p / jnp.maximum(c_ref[...], 1.0)
        h2 = (jnp.dot(agg, w2l_ref[...], preferred_element_type=jnp.float32)
              + h1r_ref[...])
        h2 = jnp.maximum(h2, 0.0)

        gids = lax.broadcasted_iota(jnp.int32, (1, n_graphs), 1)
        bm = (b_ref[...] == gids).astype(jnp.float32)
        dn = (((0,), (0,)), ((), ()))
        pool_s[...] += lax.dot_general(bm, h2, dn,
                                       preferred_element_type=jnp.float32)
        ones = jnp.ones((r, d), jnp.float32)
        gcnt_s[...] += lax.dot_general(bm, ones, dn,
                                       preferred_element_type=jnp.float32)

        rid = lax.broadcasted_iota(jnp.int32, (r, 1), 0) + i * r
        pm = (rid == pid_ref[...]).astype(jnp.float32)
        prow_s[...] += lax.dot_general(pm, h2, dn,
                                       preferred_element_type=jnp.float32)

        @pl.when(i == g_steps - 1)
        def _epilogue():
            graph_emb = pool_s[...] / jnp.maximum(gcnt_s[...], 1.0)
            flow_emb = jnp.maximum(
                jnp.dot(flow_ref[...], wf_ref[...],
                        preferred_element_type=jnp.float32) + bf_ref[...],
                0.0)
            rows = prow_s[...]
            h = jnp.zeros((n_graphs, hh), jnp.float32)
            c = jnp.zeros((n_graphs, hh), jnp.float32)
            for t in range(path_len):
                xt = rows[t * n_graphs:(t + 1) * n_graphs, :]
                gg = (jnp.dot(xt, wih_ref[...],
                              preferred_element_type=jnp.float32)
                      + bih_ref[...]
                      + jnp.dot(h, whh_ref[...],
                                preferred_element_type=jnp.float32)
                      + bhh_ref[...])
                gi = jax.nn.sigmoid(gg[:, 0 * hh:1 * hh])
                gf = jax.nn.sigmoid(gg[:, 1 * hh:2 * hh])
                gc = jnp.tanh(gg[:, 2 * hh:3 * hh])
                go = jax.nn.sigmoid(gg[:, 3 * hh:4 * hh])
                c = gf * c + gi * gc
                h = go * jnp.tanh(c)
            comb = jnp.concatenate([graph_emb, h, flow_emb], axis=1)
            hid = jnp.maximum(
                jnp.dot(comb, ws1_ref[...],
                        preferred_element_type=jnp.float32) + bs1_ref[...],
                0.0)
            o_ref[...] = (jnp.dot(hid, ws2_ref[...],
                                  preferred_element_type=jnp.float32)
                          + bs2_ref[...])

    whole = lambda shape: pl.BlockSpec(shape, lambda i: tuple(0 for _ in shape))
    return pl.pallas_call(
        body,
        grid=(g_steps,),
        in_specs=[
            pl.BlockSpec((NC, r, d), lambda i: (0, i, 0)),
            pl.BlockSpec((r, 1), lambda i: (i, 0)),
            pl.BlockSpec((r, d), lambda i: (i, 0)),
            pl.BlockSpec((r, 1), lambda i: (i, 0)),
            whole(pid2d.shape),
            whole(flow.shape),
            whole(w2lT.shape),
            whole(wihT.shape),
            whole(whhT.shape),
            whole(bihrow.shape),
            whole(bhhrow.shape),
            whole(wfT.shape),
            whole(bfrow.shape),
            whole(ws1T.shape),
            whole(bs1row.shape),
            whole(ws2T.shape),
            whole(bs2row.shape),
        ],
        out_specs=pl.BlockSpec((n_graphs, 1), lambda i: (0, 0)),
        out_shape=jax.ShapeDtypeStruct((n_graphs, 1), jnp.float32),
        scratch_shapes=[
            pltpu.VMEM((n_graphs, d), jnp.float32),
            pltpu.VMEM((n_graphs, d), jnp.float32),
            pltpu.VMEM((npath, d), jnp.float32),
        ],
        interpret=interpret,
    )(part, cnt, h1r, batch2d, pid2d, flow,
      w2lT, wihT, whhT, bihrow, bhhrow,
      wfT, bfrow, ws1T, bs1row, ws2T, bs2row)


def kernel(x, edge_index, batch, path_node_indices_list, flow_feature_batch,
           W1l, W1r, b1, W2l, W2r, b2, W_ih, W_hh, b_ih, b_hh,
           Wf, bf, Ws1, bs1, Ws2, bs2):
    n, d = x.shape
    n_graphs, path_len = path_node_indices_list.shape
    src = edge_index[0]
    dst = edge_index[1]
    e = src.shape[0]
    nch = e // (NW * K)
    src3d = src.reshape(NW, nch, K)
    dst3d = dst.reshape(NW, nch, K)

    zero_rows = jnp.zeros((K, d), jnp.float32)

    part1 = _sc_aggregate(x, src3d, dst3d, zero_rows)
    cnt = _tc_count(dst.reshape(-1, 1), n)
    cnt = cnt.reshape(-1)[:n].reshape(n, 1)
    xr = _tc_xr(x, W1r.T, b1.reshape(1, -1))

    h1 = _tc_sage(part1, cnt, xr, W1l.T)

    part2 = _sc_aggregate(h1, src3d, dst3d, zero_rows)
    h1r = _tc_xr(h1, W2r.T, b2.reshape(1, -1))

    pid2d = path_node_indices_list.T.reshape(1, -1)
    scores = _tc_final(
        part2, cnt, h1r, batch.reshape(-1, 1), pid2d, flow_feature_batch,
        W2l.T,
        W_ih.T, W_hh.T, b_ih.reshape(1, -1), b_hh.reshape(1, -1),
        Wf.T, bf.reshape(1, -1), Ws1.T, bs1.reshape(1, -1),
        Ws2.T, bs2.reshape(1, 1),
        n_graphs, path_len)
    return scores.reshape(-1)


# speedup vs baseline: 1.1193x; 1.0721x over previous
import functools

import jax
import jax.numpy as jnp
from jax import lax
from jax.experimental import pallas as pl
from jax.experimental.pallas import tpu as pltpu
from jax.experimental.pallas import tpu_sc as plsc

NC = 2
NS = 16
NW = NC * NS
K = 80


def _rows_per_tile(n):
    rpt = -(-n // NS)
    return rpt + (-rpt) % K


def _sc_aggregate(table, src3d, dst3d, zero_rows):
    n, d = table.shape
    _, nch, _ = src3d.shape
    rpt = _rows_per_tile(n)
    nrch = rpt // K

    mesh = plsc.VectorSubcoreMesh(
        core_axis_name="c", subcore_axis_name="s",
        num_cores=NC, num_subcores=NS)

    def body(table_h, src_h, dst_h, zr_h, part_h,
             acc_sh, didx, sA, sB, rA, rB,
             semA, semB, semIA, semIB, semSA, semSB):
        c = lax.axis_index("c")
        s = lax.axis_index("s")
        wid = c * NS + s
        row0 = jnp.minimum(s * rpt, n - rpt)

        pltpu.sync_copy(dst_h.at[wid], didx)

        pltpu.sync_copy(zr_h, rA)

        def zloop(j, carry):
            pltpu.sync_copy(rA, acc_sh.at[pl.ds(row0 + j * K, K)])
            return carry

        lax.fori_loop(0, nrch, zloop, 0)
        plsc.subcore_barrier()

        bufA = (sA, rA, semA, semIA, semSA)
        bufB = (sB, rB, semB, semIB, semSB)

        def idx_start(g, buf):
            s_v, _, _, semI, _ = buf
            pltpu.async_copy(src_h.at[wid, g], s_v, semI)

        def gather_start(g, buf):
            s_v, r_v, sem, semI, _ = buf
            pltpu.make_async_copy(src_h.at[wid, g], s_v, semI).wait()
            pltpu.async_copy(table_h.at[s_v], r_v, sem)

        def gather_wait(buf):
            s_v, r_v, sem, _, _ = buf
            pltpu.make_async_copy(table_h.at[s_v], r_v, sem).wait()

        def scatter(g, buf):
            _, r_v, _, _, _ = buf
            pltpu.sync_copy(r_v, acc_sh.at[didx.at[g]], add=True)

        idx_start(0, bufA)
        idx_start(1, bufB)
        gather_start(0, bufA)

        def pipe(i, carry):
            g = 1 + 2 * i
            gather_start(g, bufB)
            gather_wait(bufA)
            idx_start(g + 1, bufA)
            scatter(g - 1, bufA)
            gather_start(g + 1, bufA)
            gather_wait(bufB)

            @pl.when(g + 2 < nch)
            def _():
                idx_start(g + 2, bufB)

            scatter(g, bufB)
            return carry

        lax.fori_loop(0, (nch - 1) // 2, pipe, 0)
        gather_wait(bufA)
        scatter(nch - 1, bufA)

        plsc.subcore_barrier()

        def dump(j, carry):
            r = row0 + j * K
            pltpu.sync_copy(acc_sh.at[pl.ds(r, K)], rA)
            pltpu.sync_copy(rA, part_h.at[c, pl.ds(r, K)])
            return carry

        lax.fori_loop(0, nrch, dump, 0)

    kern = pl.kernel(
        body,
        out_type=jax.ShapeDtypeStruct((NC, n, d), jnp.float32),
        mesh=mesh,
        scratch_types=[
            pltpu.VMEM_SHARED((n, d), jnp.float32),
            pltpu.VMEM((nch, K), jnp.int32),
            pltpu.VMEM((K,), jnp.int32),
            pltpu.VMEM((K,), jnp.int32),
            pltpu.VMEM((K, d), jnp.float32),
            pltpu.VMEM((K, d), jnp.float32),
            pltpu.SemaphoreType.DMA,
            pltpu.SemaphoreType.DMA,
            pltpu.SemaphoreType.DMA,
            pltpu.SemaphoreType.DMA,
            pltpu.SemaphoreType.DMA,
            pltpu.SemaphoreType.DMA,
        ])
    return kern(table, src3d, dst3d, zero_rows)


def _tc_count(dst2d, n, interpret=False):
    e = dst2d.shape[0]
    eb = 8000
    hi_bins = -(-n // 128)

    def body(d_ref, o_ref):
        i = pl.program_id(0)

        @pl.when(i == 0)
        def _init():
            o_ref[...] = jnp.zeros_like(o_ref)

        dv = d_ref[...]
        lo = lax.rem(dv, 128)
        hi = lax.div(dv, 128)
        oh_lo = (lo == lax.broadcasted_iota(jnp.int32, (1, 128), 1)
                 ).astype(jnp.float32)
        oh_hi = (hi == lax.broadcasted_iota(jnp.int32, (1, hi_bins), 1)
                 ).astype(jnp.float32)
        dn = (((0,), (0,)), ((), ()))
        o_ref[...] += lax.dot_general(oh_hi, oh_lo, dn,
                                      preferred_element_type=jnp.float32)

    return pl.pallas_call(
        body,
        grid=(e // eb,),
        in_specs=[pl.BlockSpec((eb, 1), lambda i: (i, 0))],
        out_specs=pl.BlockSpec((hi_bins, 128), lambda i: (0, 0)),
        out_shape=jax.ShapeDtypeStruct((hi_bins, 128), jnp.float32),
        interpret=interpret,
    )(dst2d)


def _tc_sage(part, cnt, xin, wlT, wrT, brow, interpret=False):
    n, d = xin.shape
    r = 2000

    def body(p_ref, c_ref, x_ref, wl_ref, wr_ref, b_ref, o_ref):
        p = p_ref[0] + p_ref[1]
        agg = p / jnp.maximum(c_ref[...], 1.0)
        h = (jnp.dot(agg, wl_ref[...], preferred_element_type=jnp.float32)
             + jnp.dot(x_ref[...], wr_ref[...],
                       preferred_element_type=jnp.float32)
             + b_ref[...])
        o_ref[...] = jnp.maximum(h, 0.0)

    return pl.pallas_call(
        body,
        grid=(n // r,),
        in_specs=[
            pl.BlockSpec((NC, r, d), lambda i: (0, i, 0)),
            pl.BlockSpec((r, 1), lambda i: (i, 0)),
            pl.BlockSpec((r, d), lambda i: (i, 0)),
            pl.BlockSpec((d, d), lambda i: (0, 0)),
            pl.BlockSpec((d, d), lambda i: (0, 0)),
            pl.BlockSpec((1, d), lambda i: (0, 0)),
        ],
        out_specs=pl.BlockSpec((r, d), lambda i: (i, 0)),
        out_shape=jax.ShapeDtypeStruct((n, d), jnp.float32),
        interpret=interpret,
    )(part, cnt, xin, wlT, wrT, brow)


def _tc_final(part, cnt, h1, batch2d, pid2d, flow,
              w2lT, w2rT, b2row, wihT, whhT, bihrow, bhhrow,
              wfT, bfrow, ws1T, bs1row, ws2T, bs2row,
              n_graphs, path_len, interpret=False):
    n, d = h1.shape
    r = 2000
    g_steps = n // r
    npath = pid2d.shape[1]
    hh = wihT.shape[1] // 4

    def body(p_ref, c_ref, h1_ref, b_ref, pid_ref, flow_ref,
             w2l_ref, w2r_ref, b2_ref, wih_ref, whh_ref, bih_ref, bhh_ref,
             wf_ref, bf_ref, ws1_ref, bs1_ref, ws2_ref, bs2_ref,
             o_ref, pool_s, gcnt_s, prow_s):
        i = pl.program_id(0)

        @pl.when(i == 0)
        def _init():
            pool_s[...] = jnp.zeros_like(pool_s)
            gcnt_s[...] = jnp.zeros_like(gcnt_s)
            prow_s[...] = jnp.zeros_like(prow_s)

        p = p_ref[0] + p_ref[1]
        agg = p / jnp.maximum(c_ref[...], 1.0)
        h2 = (jnp.dot(agg, w2l_ref[...], preferred_element_type=jnp.float32)
              + jnp.dot(h1_ref[...], w2r_ref[...],
                        preferred_element_type=jnp.float32)
              + b2_ref[...])
        h2 = jnp.maximum(h2, 0.0)

        gids = lax.broadcasted_iota(jnp.int32, (1, n_graphs), 1)
        bm = (b_ref[...] == gids).astype(jnp.float32)
        dn = (((0,), (0,)), ((), ()))
        pool_s[...] += lax.dot_general(bm, h2, dn,
                                       preferred_element_type=jnp.float32)
        ones = jnp.ones((r, d), jnp.float32)
        gcnt_s[...] += lax.dot_general(bm, ones, dn,
                                       preferred_element_type=jnp.float32)

        rid = lax.broadcasted_iota(jnp.int32, (r, 1), 0) + i * r
        pm = (rid == pid_ref[...]).astype(jnp.float32)
        prow_s[...] += lax.dot_general(pm, h2, dn,
                                       preferred_element_type=jnp.float32)

        @pl.when(i == g_steps - 1)
        def _epilogue():
            graph_emb = pool_s[...] / jnp.maximum(gcnt_s[...], 1.0)
            flow_emb = jnp.maximum(
                jnp.dot(flow_ref[...], wf_ref[...],
                        preferred_element_type=jnp.float32) + bf_ref[...],
                0.0)
            rows = prow_s[...]
            h = jnp.zeros((n_graphs, hh), jnp.float32)
            c = jnp.zeros((n_graphs, hh), jnp.float32)
            for t in range(path_len):
                xt = rows[t * n_graphs:(t + 1) * n_graphs, :]
                gg = (jnp.dot(xt, wih_ref[...],
                              preferred_element_type=jnp.float32)
                      + bih_ref[...]
                      + jnp.dot(h, whh_ref[...],
                                preferred_element_type=jnp.float32)
                      + bhh_ref[...])
                gi = jax.nn.sigmoid(gg[:, 0 * hh:1 * hh])
                gf = jax.nn.sigmoid(gg[:, 1 * hh:2 * hh])
                gc = jnp.tanh(gg[:, 2 * hh:3 * hh])
                go = jax.nn.sigmoid(gg[:, 3 * hh:4 * hh])
                c = gf * c + gi * gc
                h = go * jnp.tanh(c)
            comb = jnp.concatenate([graph_emb, h, flow_emb], axis=1)
            hid = jnp.maximum(
                jnp.dot(comb, ws1_ref[...],
                        preferred_element_type=jnp.float32) + bs1_ref[...],
                0.0)
            o_ref[...] = (jnp.dot(hid, ws2_ref[...],
                                  preferred_element_type=jnp.float32)
                          + bs2_ref[...])

    whole = lambda shape: pl.BlockSpec(shape, lambda i: tuple(0 for _ in shape))
    return pl.pallas_call(
        body,
        grid=(g_steps,),
        in_specs=[
            pl.BlockSpec((NC, r, d), lambda i: (0, i, 0)),
            pl.BlockSpec((r, 1), lambda i: (i, 0)),
            pl.BlockSpec((r, d), lambda i: (i, 0)),
            pl.BlockSpec((r, 1), lambda i: (i, 0)),
            whole(pid2d.shape),
            whole(flow.shape),
            whole(w2lT.shape),
            whole(w2rT.shape),
            whole(b2row.shape),
            whole(wihT.shape),
            whole(whhT.shape),
            whole(bihrow.shape),
            whole(bhhrow.shape),
            whole(wfT.shape),
            whole(bfrow.shape),
            whole(ws1T.shape),
            whole(bs1row.shape),
            whole(ws2T.shape),
            whole(bs2row.shape),
        ],
        out_specs=pl.BlockSpec((n_graphs, 1), lambda i: (0, 0)),
        out_shape=jax.ShapeDtypeStruct((n_graphs, 1), jnp.float32),
        scratch_shapes=[
            pltpu.VMEM((n_graphs, d), jnp.float32),
            pltpu.VMEM((n_graphs, d), jnp.float32),
            pltpu.VMEM((npath, d), jnp.float32),
        ],
        interpret=interpret,
    )(part, cnt, h1, batch2d, pid2d, flow,
      w2lT, w2rT, b2row, wihT, whhT, bihrow, bhhrow,
      wfT, bfrow, ws1T, bs1row, ws2T, bs2row)


def kernel(x, edge_index, batch, path_node_indices_list, flow_feature_batch,
           W1l, W1r, b1, W2l, W2r, b2, W_ih, W_hh, b_ih, b_hh,
           Wf, bf, Ws1, bs1, Ws2, bs2):
    n, d = x.shape
    n_graphs, path_len = path_node_indices_list.shape
    src = edge_index[0]
    dst = edge_index[1]
    e = src.shape[0]
    nch = e // (NW * K)
    src3d = src.reshape(NW, nch, K)
    dst3d = dst.reshape(NW, nch, K)

    zero_rows = jnp.zeros((K, d), jnp.float32)

    cnt = _tc_count(dst.reshape(-1, 1), n)
    cnt = cnt.reshape(-1)[:n].reshape(n, 1)

    part1 = _sc_aggregate(x, src3d, dst3d, zero_rows)
    h1 = _tc_sage(part1, cnt, x, W1l.T, W1r.T, b1.reshape(1, -1))
    part2 = _sc_aggregate(h1, src3d, dst3d, zero_rows)

    pid2d = path_node_indices_list.T.reshape(1, -1)
    scores = _tc_final(
        part2, cnt, h1, batch.reshape(-1, 1), pid2d, flow_feature_batch,
        W2l.T, W2r.T, b2.reshape(1, -1),
        W_ih.T, W_hh.T, b_ih.reshape(1, -1), b_hh.reshape(1, -1),
        Wf.T, bf.reshape(1, -1), Ws1.T, bs1.reshape(1, -1),
        Ws2.T, bs2.reshape(1, 1),
        n_graphs, path_len)
    return scores.reshape(-1)


# speedup vs baseline: 1.1306x; 1.0101x over previous
import functools

import jax
import jax.numpy as jnp
from jax import lax
from jax.experimental import pallas as pl
from jax.experimental.pallas import tpu as pltpu
from jax.experimental.pallas import tpu_sc as plsc

NC = 2
NS = 16
NW = NC * NS
K = 80


def _rows_per_tile(n):
    rpt = -(-n // NS)
    return rpt + (-rpt) % K


def _sc_aggregate(table, src3d, dst3d, zero_rows):
    n, d = table.shape
    _, nch, _ = src3d.shape
    rpt = _rows_per_tile(n)
    nrch = rpt // K

    mesh = plsc.VectorSubcoreMesh(
        core_axis_name="c", subcore_axis_name="s",
        num_cores=NC, num_subcores=NS)

    def body(table_h, src_h, dst_h, zr_h, part_h,
             acc_sh, didx, sA, sB, rA, rB,
             semA, semB, semIA, semIB, semSA, semSB):
        c = lax.axis_index("c")
        s = lax.axis_index("s")
        wid = c * NS + s
        row0 = jnp.minimum(s * rpt, n - rpt)

        pltpu.sync_copy(dst_h.at[wid], didx)

        pltpu.sync_copy(zr_h, rA)

        def zloop(j, carry):
            pltpu.sync_copy(rA, acc_sh.at[pl.ds(row0 + j * K, K)])
            return carry

        lax.fori_loop(0, nrch, zloop, 0)
        plsc.subcore_barrier()

        bufA = (sA, rA, semA, semIA, semSA)
        bufB = (sB, rB, semB, semIB, semSB)

        def idx_start(g, buf):
            s_v, _, _, semI, _ = buf
            pltpu.async_copy(src_h.at[wid, g], s_v, semI)

        def gather_start(g, buf):
            s_v, r_v, sem, semI, _ = buf
            pltpu.make_async_copy(src_h.at[wid, g], s_v, semI).wait()
            pltpu.async_copy(table_h.at[s_v], r_v, sem)

        def gather_wait(buf):
            s_v, r_v, sem, _, _ = buf
            pltpu.make_async_copy(table_h.at[s_v], r_v, sem).wait()

        def scatter(g, buf):
            _, r_v, _, _, _ = buf
            pltpu.sync_copy(r_v, acc_sh.at[didx.at[g]], add=True)

        idx_start(0, bufA)
        idx_start(1, bufB)
        gather_start(0, bufA)

        def pipe(i, carry):
            g = 1 + 2 * i
            gather_start(g, bufB)
            gather_wait(bufA)
            idx_start(g + 1, bufA)
            scatter(g - 1, bufA)
            gather_start(g + 1, bufA)
            gather_wait(bufB)

            @pl.when(g + 2 < nch)
            def _():
                idx_start(g + 2, bufB)

            scatter(g, bufB)
            return carry

        lax.fori_loop(0, (nch - 1) // 2, pipe, 0)
        gather_wait(bufA)
        scatter(nch - 1, bufA)

        plsc.subcore_barrier()

        def dump(j, carry):
            r = row0 + j * K
            pltpu.sync_copy(acc_sh.at[pl.ds(r, K)], rA)
            pltpu.sync_copy(rA, part_h.at[c, pl.ds(r, K)])
            return carry

        lax.fori_loop(0, nrch, dump, 0)

    kern = pl.kernel(
        body,
        out_type=jax.ShapeDtypeStruct((NC, n, d), jnp.float32),
        mesh=mesh,
        scratch_types=[
            pltpu.VMEM_SHARED((n, d), jnp.float32),
            pltpu.VMEM((nch, K), jnp.int32),
            pltpu.VMEM((K,), jnp.int32),
            pltpu.VMEM((K,), jnp.int32),
            pltpu.VMEM((K, d), jnp.float32),
            pltpu.VMEM((K, d), jnp.float32),
            pltpu.SemaphoreType.DMA,
            pltpu.SemaphoreType.DMA,
            pltpu.SemaphoreType.DMA,
            pltpu.SemaphoreType.DMA,
            pltpu.SemaphoreType.DMA,
            pltpu.SemaphoreType.DMA,
        ])
    return kern(table, src3d, dst3d, zero_rows)


def _tc_count(dst2d, n, interpret=False):
    e = dst2d.shape[0]
    eb = 16000
    hi_bins = -(-n // 128)

    def body(d_ref, o_ref):
        i = pl.program_id(0)

        @pl.when(i == 0)
        def _init():
            o_ref[...] = jnp.zeros_like(o_ref)

        dv = d_ref[...]
        lo = lax.rem(dv, 128)
        hi = lax.div(dv, 128)
        oh_lo = (lo == lax.broadcasted_iota(jnp.int32, (1, 128), 1)
                 ).astype(jnp.float32)
        oh_hi = (hi == lax.broadcasted_iota(jnp.int32, (1, hi_bins), 1)
                 ).astype(jnp.float32)
        dn = (((0,), (0,)), ((), ()))
        o_ref[...] += lax.dot_general(oh_hi, oh_lo, dn,
                                      preferred_element_type=jnp.float32)

    return pl.pallas_call(
        body,
        grid=(e // eb,),
        in_specs=[pl.BlockSpec((eb, 1), lambda i: (i, 0))],
        out_specs=pl.BlockSpec((hi_bins, 128), lambda i: (0, 0)),
        out_shape=jax.ShapeDtypeStruct((hi_bins, 128), jnp.float32),
        interpret=interpret,
    )(dst2d)


def _tc_sage(part, cnt, xin, wlT, wrT, brow, interpret=False):
    n, d = xin.shape
    r = 5000

    def body(p_ref, c_ref, x_ref, wl_ref, wr_ref, b_ref, o_ref):
        p = p_ref[0] + p_ref[1]
        agg = p / jnp.maximum(c_ref[...], 1.0)
        h = (jnp.dot(agg, wl_ref[...], preferred_element_type=jnp.float32)
             + jnp.dot(x_ref[...], wr_ref[...],
                       preferred_element_type=jnp.float32)
             + b_ref[...])
        o_ref[...] = jnp.maximum(h, 0.0)

    return pl.pallas_call(
        body,
        grid=(n // r,),
        in_specs=[
            pl.BlockSpec((NC, r, d), lambda i: (0, i, 0)),
            pl.BlockSpec((r, 1), lambda i: (i, 0)),
            pl.BlockSpec((r, d), lambda i: (i, 0)),
            pl.BlockSpec((d, d), lambda i: (0, 0)),
            pl.BlockSpec((d, d), lambda i: (0, 0)),
            pl.BlockSpec((1, d), lambda i: (0, 0)),
        ],
        out_specs=pl.BlockSpec((r, d), lambda i: (i, 0)),
        out_shape=jax.ShapeDtypeStruct((n, d), jnp.float32),
        interpret=interpret,
    )(part, cnt, xin, wlT, wrT, brow)


def _tc_final(part, cnt, h1, batch2d, pid2d, flow,
              w2lT, w2rT, b2row, wihT, whhT, bihrow, bhhrow,
              wfT, bfrow, ws1T, bs1row, ws2T, bs2row,
              n_graphs, path_len, interpret=False):
    n, d = h1.shape
    r = 5000
    g_steps = n // r
    npath = pid2d.shape[1]
    hh = wihT.shape[1] // 4

    def body(p_ref, c_ref, h1_ref, b_ref, pid_ref, flow_ref,
             w2l_ref, w2r_ref, b2_ref, wih_ref, whh_ref, bih_ref, bhh_ref,
             wf_ref, bf_ref, ws1_ref, bs1_ref, ws2_ref, bs2_ref,
             o_ref, pool_s, gcnt_s, prow_s):
        i = pl.program_id(0)

        @pl.when(i == 0)
        def _init():
            pool_s[...] = jnp.zeros_like(pool_s)
            gcnt_s[...] = jnp.zeros_like(gcnt_s)
            prow_s[...] = jnp.zeros_like(prow_s)

        p = p_ref[0] + p_ref[1]
        agg = p / jnp.maximum(c_ref[...], 1.0)
        h2 = (jnp.dot(agg, w2l_ref[...], preferred_element_type=jnp.float32)
              + jnp.dot(h1_ref[...], w2r_ref[...],
                        preferred_element_type=jnp.float32)
              + b2_ref[...])
        h2 = jnp.maximum(h2, 0.0)

        gids = lax.broadcasted_iota(jnp.int32, (1, n_graphs), 1)
        bm = (b_ref[...] == gids).astype(jnp.float32)
        dn = (((0,), (0,)), ((), ()))
        pool_s[...] += lax.dot_general(bm, h2, dn,
                                       preferred_element_type=jnp.float32)
        ones = jnp.ones((r, d), jnp.float32)
        gcnt_s[...] += lax.dot_general(bm, ones, dn,
                                       preferred_element_type=jnp.float32)

        rid = lax.broadcasted_iota(jnp.int32, (r, 1), 0) + i * r
        pm = (rid == pid_ref[...]).astype(jnp.float32)
        prow_s[...] += lax.dot_general(pm, h2, dn,
                                       preferred_element_type=jnp.float32)

        @pl.when(i == g_steps - 1)
        def _epilogue():
            graph_emb = pool_s[...] / jnp.maximum(gcnt_s[...], 1.0)
            flow_emb = jnp.maximum(
                jnp.dot(flow_ref[...], wf_ref[...],
                        preferred_element_type=jnp.float32) + bf_ref[...],
                0.0)
            rows = prow_s[...]
            h = jnp.zeros((n_graphs, hh), jnp.float32)
            c = jnp.zeros((n_graphs, hh), jnp.float32)
            for t in range(path_len):
                xt = rows[t * n_graphs:(t + 1) * n_graphs, :]
                gg = (jnp.dot(xt, wih_ref[...],
                              preferred_element_type=jnp.float32)
                      + bih_ref[...]
                      + jnp.dot(h, whh_ref[...],
                                preferred_element_type=jnp.float32)
                      + bhh_ref[...])
                gi = jax.nn.sigmoid(gg[:, 0 * hh:1 * hh])
                gf = jax.nn.sigmoid(gg[:, 1 * hh:2 * hh])
                gc = jnp.tanh(gg[:, 2 * hh:3 * hh])
                go = jax.nn.sigmoid(gg[:, 3 * hh:4 * hh])
                c = gf * c + gi * gc
                h = go * jnp.tanh(c)
            comb = jnp.concatenate([graph_emb, h, flow_emb], axis=1)
            hid = jnp.maximum(
                jnp.dot(comb, ws1_ref[...],
                        preferred_element_type=jnp.float32) + bs1_ref[...],
                0.0)
            o_ref[...] = (jnp.dot(hid, ws2_ref[...],
                                  preferred_element_type=jnp.float32)
                          + bs2_ref[...])

    whole = lambda shape: pl.BlockSpec(shape, lambda i: tuple(0 for _ in shape))
    return pl.pallas_call(
        body,
        grid=(g_steps,),
        in_specs=[
            pl.BlockSpec((NC, r, d), lambda i: (0, i, 0)),
            pl.BlockSpec((r, 1), lambda i: (i, 0)),
            pl.BlockSpec((r, d), lambda i: (i, 0)),
            pl.BlockSpec((r, 1), lambda i: (i, 0)),
            whole(pid2d.shape),
            whole(flow.shape),
            whole(w2lT.shape),
            whole(w2rT.shape),
            whole(b2row.shape),
            whole(wihT.shape),
            whole(whhT.shape),
            whole(bihrow.shape),
            whole(bhhrow.shape),
            whole(wfT.shape),
            whole(bfrow.shape),
            whole(ws1T.shape),
            whole(bs1row.shape),
            whole(ws2T.shape),
            whole(bs2row.shape),
        ],
        out_specs=pl.BlockSpec((n_graphs, 1), lambda i: (0, 0)),
        out_shape=jax.ShapeDtypeStruct((n_graphs, 1), jnp.float32),
        scratch_shapes=[
            pltpu.VMEM((n_graphs, d), jnp.float32),
            pltpu.VMEM((n_graphs, d), jnp.float32),
            pltpu.VMEM((npath, d), jnp.float32),
        ],
        interpret=interpret,
    )(part, cnt, h1, batch2d, pid2d, flow,
      w2lT, w2rT, b2row, wihT, whhT, bihrow, bhhrow,
      wfT, bfrow, ws1T, bs1row, ws2T, bs2row)


def kernel(x, edge_index, batch, path_node_indices_list, flow_feature_batch,
           W1l, W1r, b1, W2l, W2r, b2, W_ih, W_hh, b_ih, b_hh,
           Wf, bf, Ws1, bs1, Ws2, bs2):
    n, d = x.shape
    n_graphs, path_len = path_node_indices_list.shape
    src = edge_index[0]
    dst = edge_index[1]
    e = src.shape[0]
    nch = e // (NW * K)
    src3d = src.reshape(NW, nch, K)
    dst3d = dst.reshape(NW, nch, K)

    zero_rows = jnp.zeros((K, d), jnp.float32)

    cnt = _tc_count(dst.reshape(-1, 1), n)
    cnt = cnt.reshape(-1)[:n].reshape(n, 1)

    part1 = _sc_aggregate(x, src3d, dst3d, zero_rows)
    h1 = _tc_sage(part1, cnt, x, W1l.T, W1r.T, b1.reshape(1, -1))
    part2 = _sc_aggregate(h1, src3d, dst3d, zero_rows)

    pid2d = path_node_indices_list.T.reshape(1, -1)
    scores = _tc_final(
        part2, cnt, h1, batch.reshape(-1, 1), pid2d, flow_feature_batch,
        W2l.T, W2r.T, b2.reshape(1, -1),
        W_ih.T, W_hh.T, b_ih.reshape(1, -1), b_hh.reshape(1, -1),
        Wf.T, bf.reshape(1, -1), Ws1.T, bs1.reshape(1, -1),
        Ws2.T, bs2.reshape(1, 1),
        n_graphs, path_len)
    return scores.reshape(-1)
